# baseline (speedup 1.0000x reference)
import functools

import jax
import jax.numpy as jnp
from jax import lax
from jax.experimental import pallas as pl
from jax.experimental.pallas import tpu as pltpu
from jax.experimental.pallas import tpu_sc as plsc

B, S, D = 4, 4096, 2048
DFF = 4 * D
K = S // 4
NTOK = B * K
TD = 512
TT = 1024
ET = 512
NE = (B * S) // ET


def _silu(z):
    return z / (1.0 + jnp.exp(-z))


_L = 16
_NW = 32
_GROWS = NTOK // _NW
_GCH = 32


def _u32key(f):
    bu = lax.bitcast_convert_type(f, jnp.uint32)
    neg = bu >= jnp.uint32(0x80000000)
    return jnp.where(neg, ~bu, bu | jnp.uint32(0x80000000))


def _sc_select_row(row, logits_hbm, gidx_hbm, rw_hbm, tar_hbm,
                   log_v, keys_v, idx_v, rwv_v, tar_v, sem):
    nch = S // _L
    pltpu.sync_copy(logits_hbm.at[row], log_v)

    def _keys(c, carry):
        f = log_v[pl.ds(c * _L, _L)]
        keys_v[pl.ds(c * _L, _L)] = _u32key(f)
        return carry
    lax.fori_loop(0, nch, _keys, 0, unroll=4)

    kk = jnp.full((_L,), K, jnp.int32)

    def _bit(i, t):
        t2 = t | (jnp.uint32(1) << (31 - i).astype(jnp.uint32))

        def _cnt(c, acc):
            kv = keys_v[pl.ds(c * _L, _L)]
            m = kv > t2
            return acc + plsc.all_reduce_population_count(m)
        cnt = lax.fori_loop(0, nch, _cnt, jnp.zeros((_L,), jnp.int32),
                            unroll=4)
        return jnp.where(cnt >= kk, t2, t)
    t = lax.fori_loop(0, 32, _bit, jnp.zeros((_L,), jnp.uint32))
    tau = t + jnp.uint32(1)

    def _cntg(c, acc):
        kv = keys_v[pl.ds(c * _L, _L)]
        return acc + plsc.all_reduce_population_count(kv > tau)
    g = lax.fori_loop(0, nch, _cntg, jnp.zeros((_L,), jnp.int32), unroll=4)
    need = kk - g

    gbase = row * S
    lane = lax.iota(jnp.int32, _L)

    def _compact(c, carry):
        off, eqc = carry
        kv = keys_v[pl.ds(c * _L, _L)]
        fv = log_v[pl.ds(c * _L, _L)]
        gt = kv > tau
        eq = kv == tau
        eqpos = eqc + plsc.cumsum(eq.astype(jnp.int32))
        sel = gt | (eq & (eqpos <= need))
        pos = off + plsc.cumsum(sel.astype(jnp.int32)) - 1
        gvec = gbase + c * _L + lane
        plsc.store_scatter(idx_v, [pos], gvec, mask=sel)
        plsc.store_scatter(rwv_v, [pos], fv, mask=sel)
        tar_v[pl.ds(c * _L, _L)] = jnp.where(sel, 1.0, 0.0)
        off = off + plsc.all_reduce_population_count(sel)
        eqc = eqc + plsc.all_reduce_population_count(eq)
        return off, eqc
    z = jnp.zeros((_L,), jnp.int32)
    lax.fori_loop(0, nch, _compact, (z, z), unroll=2)

    pltpu.sync_copy(idx_v, gidx_hbm.at[pl.ds(row * K, K)])
    pltpu.sync_copy(rwv_v, rw_hbm.at[pl.ds(row * K, K)])
    pltpu.sync_copy(tar_v, tar_hbm.at[pl.ds(row * S, S)])


def _sc_select_gather_body(logits_hbm, x_hbm,
                           gidx_hbm, rw_hbm, tar_hbm, xs_hbm,
                           log_v, keys_v, idx_v, rwv_v, tar_v,
                           idxc_v, rows_v, sem):
    c = lax.axis_index("c")
    s = lax.axis_index("s")
    wid = s * 2 + c

    @pl.when(s < B)
    def _():
        _sc_select_row(s, logits_hbm, gidx_hbm, rw_hbm, tar_hbm,
                       log_v, keys_v, idx_v, rwv_v, tar_v, sem)

    plsc.subcore_barrier()

    base = wid * _GROWS

    def _chunk(j, carry):
        st = base + j * _GCH
        pltpu.sync_copy(gidx_hbm.at[pl.ds(st, _GCH)], idxc_v)
        pltpu.async_copy(x_hbm.at[idxc_v], rows_v, sem).wait()
        pltpu.sync_copy(rows_v, xs_hbm.at[pl.ds(st, _GCH)])
        return carry
    lax.fori_loop(0, _GROWS // _GCH, _chunk, 0)


def _sc_select_gather(router_logits, x2d):
    mesh = plsc.VectorSubcoreMesh(core_axis_name="c", subcore_axis_name="s")
    f = pl.kernel(
        _sc_select_gather_body,
        out_type=[
            jax.ShapeDtypeStruct((NTOK,), jnp.int32),
            jax.ShapeDtypeStruct((NTOK,), jnp.float32),
            jax.ShapeDtypeStruct((B * S,), jnp.float32),
            jax.ShapeDtypeStruct((NTOK, D), jnp.float32),
        ],
        mesh=mesh,
        scratch_types=[
            pltpu.VMEM((S,), jnp.float32),
            pltpu.VMEM((S,), jnp.uint32),
            pltpu.VMEM((K,), jnp.int32),
            pltpu.VMEM((K,), jnp.float32),
            pltpu.VMEM((S,), jnp.float32),
            pltpu.VMEM((_GCH,), jnp.int32),
            pltpu.VMEM((_GCH, D), jnp.float32),
            pltpu.SemaphoreType.DMA,
        ],
        compiler_params=pltpu.CompilerParams(needs_layout_passes=False),
    )
    return f(router_logits, x2d)


def _sc_scatter_body(p_hbm, gidx2_hbm, pd_hbm, idxc_v, rows_v, sem):
    c = lax.axis_index("c")
    s = lax.axis_index("s")
    wid = s * 2 + c
    nch = NTOK // _GCH // _NW

    def _chunk(j, carry):
        r = wid * nch + j
        pltpu.sync_copy(gidx2_hbm.at[r], idxc_v)
        pltpu.sync_copy(p_hbm.at[pl.ds(r * _GCH, _GCH)], rows_v)
        pltpu.async_copy(rows_v, pd_hbm.at[idxc_v], sem).wait()
        return carry
    lax.fori_loop(0, nch, _chunk, 0)


def _sc_scatter(p, gidx):
    mesh = plsc.VectorSubcoreMesh(core_axis_name="c", subcore_axis_name="s")
    f = pl.kernel(
        _sc_scatter_body,
        out_type=jax.ShapeDtypeStruct((B * S, D), jnp.float32),
        mesh=mesh,
        scratch_types=[
            pltpu.VMEM((_GCH,), jnp.int32),
            pltpu.VMEM((_GCH, D), jnp.float32),
            pltpu.SemaphoreType.DMA,
        ],
        compiler_params=pltpu.CompilerParams(needs_layout_passes=False),
    )
    return f(p, gidx.reshape(NTOK // _GCH, _GCH))


def _mlp_body(rw_ref, xs_ref, w1_ref, b1_ref, w2_ref, b2_ref, out_ref, h_ref):
    j = pl.program_id(1)

    @pl.when(j == 0)
    def _():
        out_ref[...] = jnp.zeros_like(out_ref)

    xb = xs_ref[...]
    w1 = w1_ref[...].astype(jnp.bfloat16)
    z = jnp.dot(xb, w1, preferred_element_type=jnp.float32) + b1_ref[...]
    h_ref[...] = _silu(z).astype(jnp.bfloat16)
    w2 = w2_ref[...].astype(jnp.bfloat16)
    out_ref[...] += jnp.dot(h_ref[...], w2, preferred_element_type=jnp.float32)

    @pl.when(j == DFF // TD - 1)
    def _():
        out_ref[...] = (out_ref[...] + b2_ref[...]) * rw_ref[...]


def _mlp(xs_bf, rw_col, W1, b1, W2, b2):
    grid = (NTOK // TT, DFF // TD)
    return pl.pallas_call(
        _mlp_body,
        grid=grid,
        in_specs=[
            pl.BlockSpec((TT, 1), lambda i, j: (i, 0)),
            pl.BlockSpec((TT, D), lambda i, j: (i, 0)),
            pl.BlockSpec((D, TD), lambda i, j: (0, j)),
            pl.BlockSpec((1, TD), lambda i, j: (0, j)),
            pl.BlockSpec((TD, D), lambda i, j: (j, 0)),
            pl.BlockSpec((1, D), lambda i, j: (0, 0)),
        ],
        out_specs=pl.BlockSpec((TT, D), lambda i, j: (i, 0)),
        out_shape=jax.ShapeDtypeStruct((NTOK, D), jnp.float32),
        scratch_shapes=[pltpu.VMEM((TT, TD), jnp.bfloat16)],
    )(rw_col, xs_bf, W1, b1.reshape(1, DFF), W2, b2.reshape(1, D))


def _merge_aux_body(x_ref, pd_ref, m_ref, wa1_ref, wa2_ref, ba1_ref, ba2_ref,
                    out_ref, aux_ref):
    i = pl.program_id(0)
    xt = x_ref[...]
    m = m_ref[...]
    out_ref[...] = jnp.where(m > 0.5, pd_ref[...], xt)

    wa1 = wa1_ref[...].astype(jnp.bfloat16)
    a = jnp.dot(xt.astype(jnp.bfloat16), wa1, preferred_element_type=jnp.float32)
    a = _silu(a + ba1_ref[...]).astype(jnp.bfloat16)
    z = jnp.sum(a.astype(jnp.float32) * wa2_ref[...], axis=1, keepdims=True)
    z = z + ba2_ref[0, 0]
    t = m
    bce = jnp.maximum(z, 0.0) - z * t + jnp.log1p(jnp.exp(-jnp.abs(z)))
    part = jnp.sum(bce)

    @pl.when(i == 0)
    def _():
        aux_ref[0, 0] = 0.0
    aux_ref[0, 0] += part


def _merge_aux(x2d, p_dense, mask_col, Wa1, ba1, Wa2, ba2):
    grid = (NE,)
    out, aux = pl.pallas_call(
        _merge_aux_body,
        grid=grid,
        in_specs=[
            pl.BlockSpec((ET, D), lambda i: (i, 0)),
            pl.BlockSpec((ET, D), lambda i: (i, 0)),
            pl.BlockSpec((ET, 1), lambda i: (i, 0)),
            pl.BlockSpec((D, D // 2), lambda i: (0, 0)),
            pl.BlockSpec((1, D // 2), lambda i: (0, 0)),
            pl.BlockSpec((1, D // 2), lambda i: (0, 0)),
            pl.BlockSpec((1, 1), lambda i: (0, 0), memory_space=pltpu.SMEM),
        ],
        out_specs=[
            pl.BlockSpec((ET, D), lambda i: (i, 0)),
            pl.BlockSpec((1, 1), lambda i: (0, 0), memory_space=pltpu.SMEM),
        ],
        out_shape=[
            jax.ShapeDtypeStruct((B * S, D), jnp.float32),
            jax.ShapeDtypeStruct((1, 1), jnp.float32),
        ],
    )(x2d, p_dense, mask_col, Wa1, Wa2.reshape(1, D // 2),
      ba1.reshape(1, D // 2), ba2.reshape(1, 1))
    return out, aux


def kernel(x, w_router, W1, b1, W2, b2, Wa1, ba1, Wa2, ba2):
    router_logits = jnp.squeeze(x @ w_router, -1)
    x2d = x.reshape(B * S, D)

    gidx, rw, mask, xs = _sc_select_gather(router_logits, x2d)

    xs_bf = xs.astype(jnp.bfloat16)
    p = _mlp(xs_bf, rw.reshape(NTOK, 1), W1, b1, W2, b2)

    p_dense = _sc_scatter(p, gidx)

    out2d, aux = _merge_aux(x2d, p_dense, mask.reshape(B * S, 1),
                            Wa1, ba1, Wa2, ba2)
    aux_loss = aux[0, 0] / jnp.float32(B * S)
    return out2d.reshape(B, S, D), aux_loss
